# baseline (speedup 1.0000x reference)
import functools

import jax
import jax.numpy as jnp
from jax import lax
from jax.experimental import pallas as pl
from jax.experimental.pallas import tpu as pltpu
from jax.experimental.pallas import tpu_sc as plsc

_NC = 2
_NS = 16
_NW = _NC * _NS
_LANES = 16


@functools.lru_cache(maxsize=None)
def _make_detile(B, L):
    NBLK = B // 128
    BLKW = NBLK // _NW

    mesh = plsc.VectorSubcoreMesh(
        core_axis_name="c", subcore_axis_name="s",
        num_cores=_NC, num_subcores=_NS)

    @functools.partial(
        pl.kernel,
        out_type=jax.ShapeDtypeStruct((NBLK * L, 128), jnp.int32),
        mesh=mesh,
        scratch_types=[],
    )
    def detile_kernel(idxT_hbm, out_hbm):
        wid = lax.axis_index("s") * _NC + lax.axis_index("c")
        for i in range(BLKW):
            u = wid * BLKW + i
            pltpu.sync_copy(
                idxT_hbm.at[:, pl.ds(u * 128, 128)],
                out_hbm.at[pl.ds(u * L, L), :])

    return detile_kernel


@functools.lru_cache(maxsize=None)
def _make_kernel(B, L, V, D, CB):
    del V
    BPW = B // _NW
    NCH = BPW // CB
    RPC = CB * L
    assert CB == _LANES and NCH % 2 == 0

    mesh = plsc.VectorSubcoreMesh(
        core_axis_name="c", subcore_axis_name="s",
        num_cores=_NC, num_subcores=_NS)

    @functools.partial(
        pl.kernel,
        out_type=jax.ShapeDtypeStruct((B, D), jnp.float32),
        mesh=mesh,
        scratch_types=[
            pltpu.VMEM((L, CB), jnp.int32),
            pltpu.VMEM((L, CB), jnp.int32),
            pltpu.VMEM((RPC,), jnp.int32),
            pltpu.VMEM((RPC,), jnp.int32),
            pltpu.VMEM((RPC, D), jnp.float32),
            pltpu.VMEM((RPC, D), jnp.float32),
            pltpu.VMEM((BPW,), jnp.int32),
            pltpu.VMEM((BPW,), jnp.float32),
            pltpu.VMEM((BPW, D), jnp.float32),
            pltpu.SemaphoreType.DMA,
            pltpu.SemaphoreType.DMA,
            pltpu.SemaphoreType.DMA,
            pltpu.SemaphoreType.DMA,
        ],
        compiler_params=pltpu.CompilerParams(use_tc_tiling_on_sc=False),
    )
    def emb_kernel(idx2d_hbm, len_hbm, w_hbm, out_hbm,
                   idx0, idx1, fidx0, fidx1, rows0, rows1,
                   lens_v, inv_v, out_v,
                   sem_i0, sem_i1, sem_r0, sem_r1):
        wid = lax.axis_index("s") * _NC + lax.axis_index("c")
        base = wid * BPW

        pltpu.sync_copy(len_hbm.at[pl.ds(base, BPW)], lens_v)

        def inv_body(g, carry):
            lv = lens_v[pl.ds(g * _LANES, _LANES)]
            inv_v[pl.ds(g * _LANES, _LANES)] = 1.0 / lv.astype(jnp.float32)
            return carry

        lax.fori_loop(0, BPW // _LANES, inv_body, 0)

        idx_bufs = (idx0, idx1)
        fidx_bufs = (fidx0, fidx1)
        row_bufs = (rows0, rows1)
        idx_sems = (sem_i0, sem_i1)
        row_sems = (sem_r0, sem_r1)

        def start_idx(c, p):
            b0 = base + c * CB
            u = b0 // 128
            boff = b0 % 128
            pltpu.async_copy(
                idx2d_hbm.at[pl.ds(u * L, L), pl.ds(boff, CB)],
                idx_bufs[p], idx_sems[p])

        def wait_idx(p):
            pltpu.make_async_copy(
                idx2d_hbm.at[pl.ds(0, L), pl.ds(0, CB)],
                idx_bufs[p], idx_sems[p]).wait()

        def flatten_idx(p):
            src, dst = idx_bufs[p], fidx_bufs[p]

            def fbody(l, carry):
                dst[pl.ds(l * CB, CB)] = src[l, :]
                return carry

            lax.fori_loop(0, L, fbody, 0)

        def start_gather(p):
            pltpu.async_copy(
                w_hbm.at[fidx_bufs[p]], row_bufs[p], row_sems[p])

        def wait_gather(p):
            pltpu.make_async_copy(
                w_hbm.at[fidx_bufs[p]], row_bufs[p], row_sems[p]).wait()

        def process(c, p):
            wait_gather(p)

            @pl.when(c + 2 < NCH)
            def _():
                start_idx(c + 2, p)

            @pl.when(c + 1 < NCH)
            def _():
                wait_idx(1 - p)
                flatten_idx(1 - p)
                start_gather(1 - p)

            rows = row_bufs[p]
            for b in range(CB):
                def jbody(j, accs):
                    a0, a1, a2, a3 = accs
                    r = (j * 4) * CB + b
                    a0 = a0 + rows[r, :]
                    a1 = a1 + rows[r + CB, :]
                    a2 = a2 + rows[r + 2 * CB, :]
                    a3 = a3 + rows[r + 3 * CB, :]
                    return (a0, a1, a2, a3)

                z = jnp.zeros((_LANES,), jnp.float32)
                a0, a1, a2, a3 = lax.fori_loop(0, L // 4, jbody, (z, z, z, z))
                acc = (a0 + a1) + (a2 + a3)
                bb = c * CB + b
                iv = inv_v[pl.ds(c * CB, _LANES)]
                lane = jnp.full((_LANES, 1), b, jnp.int32)
                cvec = lax.gather(
                    iv, lane,
                    dimension_numbers=lax.GatherDimensionNumbers(
                        offset_dims=(), collapsed_slice_dims=(0,),
                        start_index_map=(0,)),
                    slice_sizes=(1,),
                    mode=lax.GatherScatterMode.PROMISE_IN_BOUNDS)
                out_v[bb, :] = acc * cvec

        start_idx(0, 0)
        start_idx(1, 1)
        wait_idx(0)
        flatten_idx(0)
        start_gather(0)

        def chunk_pair(g, carry):
            process(2 * g, 0)
            process(2 * g + 1, 1)
            return carry

        lax.fori_loop(0, NCH // 2, chunk_pair, 0)

        pltpu.sync_copy(out_v, out_hbm.at[pl.ds(base, BPW)])

    return emb_kernel


def kernel(input_, src_lengths, weight):
    B, L = input_.shape
    V, D = weight.shape
    idx2d = _make_detile(B, L)(input_.T)
    k = _make_kernel(B, L, V, D, CB=16)
    return k(idx2d, src_lengths, weight)


# speedup vs baseline: 1.3832x; 1.3832x over previous
import functools

import jax
import jax.numpy as jnp
from jax import lax
from jax.experimental import pallas as pl
from jax.experimental.pallas import tpu as pltpu
from jax.experimental.pallas import tpu_sc as plsc

_NC = 2
_NS = 16
_NW = _NC * _NS
_LANES = 16


@functools.lru_cache(maxsize=None)
def _make_detile(B, L):
    NBLK = B // 128

    return pl.pallas_call(
        lambda i_ref, o_ref: o_ref.__setitem__((...,), i_ref[...]),
        grid=(NBLK,),
        in_specs=[pl.BlockSpec((L, 128), lambda j: (0, j))],
        out_specs=pl.BlockSpec((L, 128), lambda j: (j, 0)),
        out_shape=jax.ShapeDtypeStruct((NBLK * L, 128), jnp.int32),
    )


_WCW = 8192
_WGRP = 1024


def _wperm(v):
    blk = jax.lax.shift_right_logical(v, 13)
    i = jax.lax.shift_left(v & 1023, 3)
    a = jax.lax.shift_right_logical(v & 8191, 10)
    return jax.lax.shift_left(blk, 13) + i + a


@functools.lru_cache(maxsize=None)
def _make_wrelayout(V, D):
    assert D == 16 and _WCW == 8 * _WGRP
    nblk = (V + _WCW - 1) // _WCW

    def body(i_ref, o_ref):
        y = i_ref[...].T
        o_ref[...] = jnp.concatenate(
            [y[a * _WGRP:(a + 1) * _WGRP, :] for a in range(8)], axis=1)

    return pl.pallas_call(
        body,
        grid=(nblk,),
        in_specs=[pl.BlockSpec((D, _WCW), lambda j: (0, j))],
        out_specs=pl.BlockSpec((_WGRP, 8 * D), lambda j: (j, 0)),
        out_shape=jax.ShapeDtypeStruct((nblk * _WGRP, 8 * D), jnp.float32),
    )


@functools.lru_cache(maxsize=None)
def _make_kernel(B, L, V, D, CB):
    del V
    BPW = B // _NW
    NCH = BPW // CB
    RPC = CB * L
    assert CB == _LANES and NCH % 2 == 0

    mesh = plsc.VectorSubcoreMesh(
        core_axis_name="c", subcore_axis_name="s",
        num_cores=_NC, num_subcores=_NS)

    @functools.partial(
        pl.kernel,
        out_type=jax.ShapeDtypeStruct((B, D), jnp.float32),
        mesh=mesh,
        scratch_types=[
            pltpu.VMEM((L, CB), jnp.int32),
            pltpu.VMEM((L, CB), jnp.int32),
            pltpu.VMEM((RPC,), jnp.int32),
            pltpu.VMEM((RPC,), jnp.int32),
            pltpu.VMEM((RPC, D), jnp.float32),
            pltpu.VMEM((RPC, D), jnp.float32),
            pltpu.VMEM((BPW,), jnp.int32),
            pltpu.VMEM((BPW,), jnp.float32),
            pltpu.VMEM((BPW, D), jnp.float32),
            pltpu.SemaphoreType.DMA,
            pltpu.SemaphoreType.DMA,
            pltpu.SemaphoreType.DMA,
            pltpu.SemaphoreType.DMA,
        ],
        compiler_params=pltpu.CompilerParams(use_tc_tiling_on_sc=False),
    )
    def emb_kernel(idx2d_hbm, len_hbm, w_hbm, out_hbm,
                   idx0, idx1, fidx0, fidx1, rows0, rows1,
                   lens_v, inv_v, out_v,
                   sem_i0, sem_i1, sem_r0, sem_r1):
        wid = lax.axis_index("s") * _NC + lax.axis_index("c")
        base = wid * BPW

        pltpu.sync_copy(len_hbm.at[pl.ds(base, BPW)], lens_v)

        def inv_body(g, carry):
            lv = lens_v[pl.ds(g * _LANES, _LANES)]
            inv_v[pl.ds(g * _LANES, _LANES)] = 1.0 / lv.astype(jnp.float32)
            return carry

        lax.fori_loop(0, BPW // _LANES, inv_body, 0)

        idx_bufs = (idx0, idx1)
        fidx_bufs = (fidx0, fidx1)
        row_bufs = (rows0, rows1)
        idx_sems = (sem_i0, sem_i1)
        row_sems = (sem_r0, sem_r1)

        def start_idx(c, p):
            b0 = base + c * CB
            u = b0 // 128
            boff = b0 % 128
            pltpu.async_copy(
                idx2d_hbm.at[pl.ds(u * L, L), pl.ds(boff, CB)],
                idx_bufs[p], idx_sems[p])

        def wait_idx(p):
            pltpu.make_async_copy(
                idx2d_hbm.at[pl.ds(0, L), pl.ds(0, CB)],
                idx_bufs[p], idx_sems[p]).wait()

        def flatten_idx(p):
            src, dst = idx_bufs[p], fidx_bufs[p]

            def fbody(l, carry):
                dst[pl.ds(l * CB, CB)] = _wperm(src[l, :])
                return carry

            lax.fori_loop(0, L, fbody, 0)

        def start_gather(p):
            pltpu.async_copy(
                w_hbm.at[fidx_bufs[p]], row_bufs[p], row_sems[p])

        def wait_gather(p):
            pltpu.make_async_copy(
                w_hbm.at[fidx_bufs[p]], row_bufs[p], row_sems[p]).wait()

        def process(c, p):
            wait_gather(p)

            @pl.when(c + 2 < NCH)
            def _():
                start_idx(c + 2, p)

            @pl.when(c + 1 < NCH)
            def _():
                wait_idx(1 - p)
                flatten_idx(1 - p)
                start_gather(1 - p)

            rows = row_bufs[p]
            for b in range(CB):
                def jbody(j, accs):
                    a0, a1, a2, a3 = accs
                    r = (j * 4) * CB + b
                    a0 = a0 + rows[r, :]
                    a1 = a1 + rows[r + CB, :]
                    a2 = a2 + rows[r + 2 * CB, :]
                    a3 = a3 + rows[r + 3 * CB, :]
                    return (a0, a1, a2, a3)

                z = jnp.zeros((_LANES,), jnp.float32)
                a0, a1, a2, a3 = lax.fori_loop(0, L // 4, jbody, (z, z, z, z))
                acc = (a0 + a1) + (a2 + a3)
                bb = c * CB + b
                iv = inv_v[pl.ds(c * CB, _LANES)]
                lane = jnp.full((_LANES, 1), b, jnp.int32)
                cvec = lax.gather(
                    iv, lane,
                    dimension_numbers=lax.GatherDimensionNumbers(
                        offset_dims=(), collapsed_slice_dims=(0,),
                        start_index_map=(0,)),
                    slice_sizes=(1,),
                    mode=lax.GatherScatterMode.PROMISE_IN_BOUNDS)
                out_v[bb, :] = acc * cvec

        start_idx(0, 0)
        start_idx(1, 1)
        wait_idx(0)
        flatten_idx(0)
        start_gather(0)

        def chunk_pair(g, carry):
            process(2 * g, 0)
            process(2 * g + 1, 1)
            return carry

        lax.fori_loop(0, NCH // 2, chunk_pair, 0)

        pltpu.sync_copy(out_v, out_hbm.at[pl.ds(base, BPW)])

    return emb_kernel


def kernel(input_, src_lengths, weight):
    B, L = input_.shape
    V, D = weight.shape
    idx2d = _make_detile(B, L)(input_.T)
    w2d = _make_wrelayout(V, D)(weight.T)
    vp = w2d.shape[0] * w2d.shape[1] // D
    k = _make_kernel(B, L, vp, D, CB=16)
    return k(idx2d, src_lengths, w2d.reshape(vp, D))
